# baseline (speedup 1.0000x reference)
import functools

import jax
import jax.numpy as jnp
from jax import lax
from jax.experimental import pallas as pl
from jax.experimental.pallas import tpu as pltpu
from jax.experimental.pallas import tpu_sc as plsc

N = 320000
D = 128
H = 256
S = 10000

_NC = 2
_NS = 16
_NW = _NC * _NS
_RB = 128
_NBLK = N // _RB
_SP = 10112
_SLAB = _SP // _NS
_NBLK_PAD = 2560
_CHUNK_A = 1280
_WBLK_A = 40
_WBLK_B = 40


def _mlp_body(x_ref, w1_ref, b1_ref, w2_ref, b2_ref, out_ref):
    x = x_ref[...].astype(jnp.bfloat16)
    h = jnp.dot(x, w1_ref[...].astype(jnp.bfloat16),
                preferred_element_type=jnp.float32)
    h = jnp.maximum(h + b1_ref[...], 0.0).astype(jnp.bfloat16)
    o = jnp.dot(h, w2_ref[...].astype(jnp.bfloat16),
                preferred_element_type=jnp.float32)
    out_ref[...] = o + b2_ref[...]


def _combine_mlp_body(pa_ref, pb_ref, pc_ref, w1_ref, b1_ref, w2_ref, b2_ref,
                      out_ref):
    seg = ((pa_ref[0] + pa_ref[1]) + (pb_ref[0] + pb_ref[1])
           + (pc_ref[0] + pc_ref[1]))
    h = jnp.dot(seg, w1_ref[...], preferred_element_type=jnp.float32)
    h = jnp.maximum(h + b1_ref[...], 0.0)
    o = jnp.dot(h, w2_ref[...], preferred_element_type=jnp.float32)
    out_ref[...] = o + b2_ref[...]


def _point_net(x, w1, b1, w2, b2, off_blocks, out_rows, block_rows=20480):
    grid = (pl.cdiv(out_rows, block_rows),)
    return pl.pallas_call(
        _mlp_body,
        grid=grid,
        in_specs=[
            pl.BlockSpec((block_rows, D), lambda i: (i + off_blocks, 0)),
            pl.BlockSpec((D, H), lambda i: (0, 0)),
            pl.BlockSpec((H,), lambda i: (0,)),
            pl.BlockSpec((H, D), lambda i: (0, 0)),
            pl.BlockSpec((D,), lambda i: (0,)),
        ],
        out_specs=pl.BlockSpec((block_rows, D), lambda i: (i, 0)),
        out_shape=jax.ShapeDtypeStruct((out_rows, D), jnp.float32),
    )(x, w1, b1, w2, b2)


def _reduce_net(pa, pb, pc, w1, b1, w2, b2, block_rows=2000):
    grid = (S // block_rows,)
    return pl.pallas_call(
        _combine_mlp_body,
        grid=grid,
        in_specs=[
            pl.BlockSpec((2, block_rows, D), lambda i: (0, i, 0)),
            pl.BlockSpec((2, block_rows, D), lambda i: (0, i, 0)),
            pl.BlockSpec((2, block_rows, D), lambda i: (0, i, 0)),
            pl.BlockSpec((D, H), lambda i: (0, 0)),
            pl.BlockSpec((H,), lambda i: (0,)),
            pl.BlockSpec((H, D), lambda i: (0, 0)),
            pl.BlockSpec((D,), lambda i: (0,)),
        ],
        out_specs=pl.BlockSpec((block_rows, D), lambda i: (i, 0)),
        out_shape=jax.ShapeDtypeStruct((S, D), jnp.float32),
    )(pa, pb, pc, w1, b1, w2, b2)


def _segment_sum_sc_body(nblk_chunk, idx_off, wblk, pt_hbm, idx_hbm,
                         zeros_hbm, out_hbm, bufs, idxs, sems_l, sems_s,
                         acc_sh):
    c = lax.axis_index("c")
    s = lax.axis_index("s")
    wid = c * _NS + s
    start = wid * wblk
    count = jnp.clip(nblk_chunk - start, 0, wblk)

    pltpu.sync_copy(zeros_hbm, acc_sh.at[pl.ds(s * _SLAB, _SLAB)])
    plsc.subcore_barrier()

    def load(blk, b):
        g = idx_off + start + blk
        pltpu.async_copy(idx_hbm.at[g], idxs[b], sems_l[b])
        pltpu.async_copy(pt_hbm.at[pl.ds((start + blk) * _RB, _RB)],
                         bufs[b], sems_l[b])

    def drain_load(b):
        pltpu.make_async_copy(idx_hbm.at[0], idxs[b], sems_l[b]).wait()
        pltpu.make_async_copy(pt_hbm.at[pl.ds(0, _RB)], bufs[b],
                              sems_l[b]).wait()

    def scat(b):
        pltpu.async_copy(bufs[b], acc_sh.at[idxs[b]], sems_s[b], add=True)

    def drain_scat(b):
        pltpu.make_async_copy(bufs[b], acc_sh.at[idxs[b]], sems_s[b]).wait()

    @pl.when(count > 0)
    def _():
        load(0, 0)

    def body(r, carry):
        for b in range(3):
            k = 3 * r + b

            @pl.when(k < count)
            def _():
                drain_load(b)
                scat(b)

                @pl.when(k >= 2)
                def _():
                    drain_scat((b + 1) % 3)

                @pl.when(k + 1 < count)
                def _():
                    load(k + 1, (b + 1) % 3)
        return carry

    lax.fori_loop(0, (count + 2) // 3, body, 0)

    for b in range(3):
        @pl.when(jnp.logical_and(count > 0, (count - 2) % 3 == b))
        def _(b=b):
            drain_scat(b)

        @pl.when(jnp.logical_and(count > 0, (count - 1) % 3 == b))
        def _(b=b):
            drain_scat(b)

    plsc.subcore_barrier()

    pltpu.sync_copy(acc_sh.at[pl.ds(s * _SLAB, _SLAB)],
                    out_hbm.at[c, pl.ds(s * _SLAB, _SLAB)])


def _segment_sum_sc(pt, idx2d, zeros_slab, nblk_chunk, idx_off, wblk):
    mesh = plsc.VectorSubcoreMesh(core_axis_name="c", subcore_axis_name="s")
    body = functools.partial(_segment_sum_sc_body, nblk_chunk, idx_off, wblk)
    k = pl.kernel(
        body,
        out_type=jax.ShapeDtypeStruct((_NC, _SP, D), jnp.float32),
        mesh=mesh,
        scratch_types=[
            [pltpu.VMEM((_RB, D), jnp.float32) for _ in range(3)],
            [pltpu.VMEM((_RB,), jnp.int32) for _ in range(3)],
            [pltpu.SemaphoreType.DMA for _ in range(3)],
            [pltpu.SemaphoreType.DMA for _ in range(3)],
            pltpu.VMEM_SHARED((_SP, D), jnp.float32),
        ],
    )
    return k(pt, idx2d, zeros_slab)


def kernel(x, idx, W1p, b1p, W2p, b2p, W1r, b1r, W2r, b2r):
    idx2d = idx.astype(jnp.int32).reshape(_NBLK, _RB)
    idx2d = jnp.pad(idx2d, ((0, _NBLK_PAD - _NBLK), (0, 0)))
    zeros_slab = jnp.zeros((_SLAB, D), jnp.float32)

    c1, c2 = 768, 768
    c3 = _NBLK - c1 - c2
    r1, r2 = c1 * _RB, c2 * _RB
    pt_a = _point_net(x, W1p, b1p, W2p, b2p, 0, r1, block_rows=16384)
    parts_a = _segment_sum_sc(pt_a, idx2d, zeros_slab, c1, 0, 24)
    pt_b = _point_net(x, W1p, b1p, W2p, b2p, r1 // 16384, r2,
                      block_rows=16384)
    parts_b = _segment_sum_sc(pt_b, idx2d, zeros_slab, c2, c1, 24)
    pt_c = _point_net(x, W1p, b1p, W2p, b2p, (r1 + r2) // 16384,
                      N - r1 - r2, block_rows=16384)
    parts_c = _segment_sum_sc(pt_c, idx2d, zeros_slab, c3, c1 + c2, 32)
    return _reduce_net(parts_a, parts_b, parts_c, W1r, b1r, W2r, b2r)


# speedup vs baseline: 1.0436x; 1.0436x over previous
import functools

import jax
import jax.numpy as jnp
from jax import lax
from jax.experimental import pallas as pl
from jax.experimental.pallas import tpu as pltpu
from jax.experimental.pallas import tpu_sc as plsc

N = 320000
D = 128
H = 256
S = 10000

_NC = 2
_NS = 16
_NW = _NC * _NS
_RB = 128
_NBLK = N // _RB
_SP = 10112
_SLAB = _SP // _NS
_NBLK_PAD = 2560
_CHUNK_A = 1280
_WBLK_A = 40
_WBLK_B = 40


def _mlp_body(x_ref, w1_ref, b1_ref, w2_ref, b2_ref, out_ref):
    x = x_ref[...].astype(jnp.bfloat16)
    h = jnp.dot(x, w1_ref[...].astype(jnp.bfloat16),
                preferred_element_type=jnp.float32)
    h = jnp.maximum(h + b1_ref[...], 0.0).astype(jnp.bfloat16)
    o = jnp.dot(h, w2_ref[...].astype(jnp.bfloat16),
                preferred_element_type=jnp.float32)
    out_ref[...] = o + b2_ref[...]


def _combine_mlp_body(pa_ref, pb_ref, w1_ref, b1_ref, w2_ref, b2_ref, out_ref):
    seg = (pa_ref[0] + pa_ref[1]) + (pb_ref[0] + pb_ref[1])
    h = jnp.dot(seg, w1_ref[...], preferred_element_type=jnp.float32)
    h = jnp.maximum(h + b1_ref[...], 0.0)
    o = jnp.dot(h, w2_ref[...], preferred_element_type=jnp.float32)
    out_ref[...] = o + b2_ref[...]


def _point_net(x, w1, b1, w2, b2, off_blocks, out_rows, block_rows=20480):
    grid = (pl.cdiv(out_rows, block_rows),)
    return pl.pallas_call(
        _mlp_body,
        grid=grid,
        in_specs=[
            pl.BlockSpec((block_rows, D), lambda i: (i + off_blocks, 0)),
            pl.BlockSpec((D, H), lambda i: (0, 0)),
            pl.BlockSpec((H,), lambda i: (0,)),
            pl.BlockSpec((H, D), lambda i: (0, 0)),
            pl.BlockSpec((D,), lambda i: (0,)),
        ],
        out_specs=pl.BlockSpec((block_rows, D), lambda i: (i, 0)),
        out_shape=jax.ShapeDtypeStruct((out_rows, D), jnp.float32),
    )(x, w1, b1, w2, b2)


def _reduce_net(pa, pb, w1, b1, w2, b2, block_rows=5000):
    grid = (S // block_rows,)
    return pl.pallas_call(
        _combine_mlp_body,
        grid=grid,
        in_specs=[
            pl.BlockSpec((2, block_rows, D), lambda i: (0, i, 0)),
            pl.BlockSpec((2, block_rows, D), lambda i: (0, i, 0)),
            pl.BlockSpec((D, H), lambda i: (0, 0)),
            pl.BlockSpec((H,), lambda i: (0,)),
            pl.BlockSpec((H, D), lambda i: (0, 0)),
            pl.BlockSpec((D,), lambda i: (0,)),
        ],
        out_specs=pl.BlockSpec((block_rows, D), lambda i: (i, 0)),
        out_shape=jax.ShapeDtypeStruct((S, D), jnp.float32),
    )(pa, pb, w1, b1, w2, b2)


def _segment_sum_sc_body(nblk_chunk, idx_off, wblk, pt_hbm, idx_hbm,
                         zeros_hbm, out_hbm, bufs, idxs, sems_l, sems_s,
                         acc_sh):
    c = lax.axis_index("c")
    s = lax.axis_index("s")
    wid = c * _NS + s
    start = wid * wblk
    count = jnp.clip(nblk_chunk - start, 0, wblk)

    pltpu.sync_copy(zeros_hbm, acc_sh.at[pl.ds(s * _SLAB, _SLAB)])
    plsc.subcore_barrier()

    def load(blk, b):
        g = idx_off + start + blk
        pltpu.async_copy(idx_hbm.at[g], idxs[b], sems_l[b])
        pltpu.async_copy(pt_hbm.at[pl.ds((start + blk) * _RB, _RB)],
                         bufs[b], sems_l[b])

    def drain_load(b):
        pltpu.make_async_copy(idx_hbm.at[0], idxs[b], sems_l[b]).wait()
        pltpu.make_async_copy(pt_hbm.at[pl.ds(0, _RB)], bufs[b],
                              sems_l[b]).wait()

    def scat(b):
        pltpu.async_copy(bufs[b], acc_sh.at[idxs[b]], sems_s[b], add=True)

    def drain_scat(b):
        pltpu.make_async_copy(bufs[b], acc_sh.at[idxs[b]], sems_s[b]).wait()

    @pl.when(count > 0)
    def _():
        load(0, 0)

    def body(r, carry):
        for b in range(3):
            k = 3 * r + b

            @pl.when(k < count)
            def _():
                drain_load(b)
                scat(b)

                @pl.when(k >= 2)
                def _():
                    drain_scat((b + 1) % 3)

                @pl.when(k + 1 < count)
                def _():
                    load(k + 1, (b + 1) % 3)
        return carry

    lax.fori_loop(0, (count + 2) // 3, body, 0)

    for b in range(3):
        @pl.when(jnp.logical_and(count > 0, (count - 2) % 3 == b))
        def _(b=b):
            drain_scat(b)

        @pl.when(jnp.logical_and(count > 0, (count - 1) % 3 == b))
        def _(b=b):
            drain_scat(b)

    plsc.subcore_barrier()

    pltpu.sync_copy(acc_sh.at[pl.ds(s * _SLAB, _SLAB)],
                    out_hbm.at[c, pl.ds(s * _SLAB, _SLAB)])


def _segment_sum_sc(pt, idx2d, zeros_slab, nblk_chunk, idx_off, wblk):
    mesh = plsc.VectorSubcoreMesh(core_axis_name="c", subcore_axis_name="s")
    body = functools.partial(_segment_sum_sc_body, nblk_chunk, idx_off, wblk)
    k = pl.kernel(
        body,
        out_type=jax.ShapeDtypeStruct((_NC, _SP, D), jnp.float32),
        mesh=mesh,
        scratch_types=[
            [pltpu.VMEM((_RB, D), jnp.float32) for _ in range(3)],
            [pltpu.VMEM((_RB,), jnp.int32) for _ in range(3)],
            [pltpu.SemaphoreType.DMA for _ in range(3)],
            [pltpu.SemaphoreType.DMA for _ in range(3)],
            pltpu.VMEM_SHARED((_SP, D), jnp.float32),
        ],
    )
    return k(pt, idx2d, zeros_slab)


def kernel(x, idx, W1p, b1p, W2p, b2p, W1r, b1r, W2r, b2r):
    idx2d = idx.astype(jnp.int32).reshape(_NBLK, _RB)
    idx2d = jnp.pad(idx2d, ((0, _NBLK_PAD - _NBLK), (0, 0)))
    zeros_slab = jnp.zeros((_SLAB, D), jnp.float32)

    rows_a = _CHUNK_A * _RB
    rows_b = N - rows_a
    pt_a = _point_net(x, W1p, b1p, W2p, b2p, 0, rows_a)
    parts_a = _segment_sum_sc(pt_a, idx2d, zeros_slab, _CHUNK_A, 0, _WBLK_A)
    pt_b = _point_net(x, W1p, b1p, W2p, b2p, rows_a // 20480, rows_b)
    parts_b = _segment_sum_sc(pt_b, idx2d, zeros_slab,
                              _NBLK - _CHUNK_A, _CHUNK_A, _WBLK_B)
    return _reduce_net(parts_a, parts_b, W1r, b1r, W2r, b2r)
